# baseline (speedup 1.0000x reference)
import functools

import jax
import jax.numpy as jnp
from jax import lax
from jax.experimental import pallas as pl
from jax.experimental.pallas import tpu as pltpu
from jax.experimental.pallas import tpu_sc as plsc

DIM = 64
L_SEQ = 200
CHUNK = 100
NCHUNK = L_SEQ // CHUNK
EPS = 1e-5
NVREG = DIM // 16


def _make_pooling(B):
    info = plsc.get_sparse_core_info()
    NC, NS = info.num_cores, info.num_subcores
    NW = NC * NS
    assert B % NW == 0
    b_per_w = B // NW
    assert b_per_w % 2 == 0
    mesh = plsc.VectorSubcoreMesh(core_axis_name="c", subcore_axis_name="s")

    @functools.partial(
        pl.kernel,
        mesh=mesh,
        compiler_params=pltpu.CompilerParams(use_tc_tiling_on_sc=False),
        out_type=jax.ShapeDtypeStruct((B, DIM), jnp.float32),
        scratch_types=[
            pltpu.VMEM((b_per_w, NCHUNK, CHUNK), jnp.int32),
            pltpu.VMEM((2, L_SEQ, DIM), jnp.float32),
            pltpu.VMEM((b_per_w, DIM), jnp.float32),
            pltpu.SemaphoreType.DMA,
            pltpu.SemaphoreType.DMA,
        ],
    )
    def pool_kernel(x_hbm, table_hbm, out_hbm, idx_v, rows_v, pooled_v,
                    sem0, sem1):
        wid = lax.axis_index("s") * NC + lax.axis_index("c")
        pltpu.sync_copy(x_hbm.at[wid], idx_v)

        sems = (sem0, sem1)

        def issue(r, b):
            for ch in range(NCHUNK):
                pltpu.make_async_copy(
                    table_hbm.at[idx_v.at[r, ch]],
                    rows_v.at[b, pl.ds(ch * CHUNK, CHUNK)],
                    sems[b],
                ).start()

        def wait(r, b):
            for ch in range(NCHUNK):
                pltpu.make_async_copy(
                    table_hbm.at[idx_v.at[r, ch]],
                    rows_v.at[b, pl.ds(ch * CHUNK, CHUNK)],
                    sems[b],
                ).wait()

        def reduce_row(r, b):
            rows = rows_v.at[b]

            def body(j, accs):
                return tuple(
                    accs[k] + rows[j, pl.ds(k * 16, 16)] for k in range(NVREG)
                )

            init = tuple(
                jnp.zeros((16,), jnp.float32) for _ in range(NVREG)
            )
            accs = lax.fori_loop(0, L_SEQ, body, init, unroll=8)
            scale = jnp.float32(1.0 / L_SEQ)
            for k in range(NVREG):
                pooled_v[r, pl.ds(k * 16, 16)] = accs[k] * scale

        issue(0, 0)

        def outer(g, carry):
            for b in range(2):
                r = g * 2 + b
                nxt = r + 1

                @pl.when(nxt < b_per_w)
                def _():
                    issue(nxt, 1 - b)

                wait(r, b)
                reduce_row(r, b)
            return carry

        lax.fori_loop(0, b_per_w // 2, outer, 0)
        pltpu.sync_copy(pooled_v, out_hbm.at[pl.ds(wid * b_per_w, b_per_w)])

    return pool_kernel


def _tail_kernel(p_ref, wt_ref, b_ref, g_ref, be_ref, o_ref):
    p = p_ref[...]
    h = jnp.dot(p, wt_ref[...], preferred_element_type=jnp.float32)
    h = h + b_ref[...]
    n = jnp.float32(1.0 / p.shape[0])
    mu = jnp.sum(h, axis=0, keepdims=True) * n
    d = h - mu
    var = jnp.sum(d * d, axis=0, keepdims=True) * n
    hn = d * lax.rsqrt(var + EPS) * g_ref[...] + be_ref[...]
    m = jnp.float32(1.0 / p.shape[1])
    mu2 = jnp.sum(hn, axis=1, keepdims=True) * m
    d2 = hn - mu2
    var2 = jnp.sum(d2 * d2, axis=1, keepdims=True) * m
    o_ref[...] = d2 * lax.rsqrt(var2 + EPS)


@jax.jit
def kernel(x, table, W, b, gamma, beta):
    B = x.shape[0]
    info = plsc.get_sparse_core_info()
    NW = info.num_cores * info.num_subcores
    x_blocks = x.reshape(NW, B // NW, NCHUNK, CHUNK)
    pooled = _make_pooling(B)(x_blocks, table)
    return pl.pallas_call(
        _tail_kernel,
        out_shape=jax.ShapeDtypeStruct((B, DIM), jnp.float32),
    )(pooled, W.T, b.reshape(1, DIM), gamma.reshape(1, DIM),
      beta.reshape(1, DIM))


# speedup vs baseline: 1.0028x; 1.0028x over previous
import functools

import jax
import jax.numpy as jnp
from jax import lax
from jax.experimental import pallas as pl
from jax.experimental.pallas import tpu as pltpu
from jax.experimental.pallas import tpu_sc as plsc

DIM = 64
L_SEQ = 200
L_PAD = 256
CHUNKS = ((0, 128), (128, 72))
EPS = 1e-5
NVREG = DIM // 16


def _make_pooling(B):
    info = plsc.get_sparse_core_info()
    NC, NS = info.num_cores, info.num_subcores
    NW = NC * NS
    assert B % NW == 0
    b_per_w = B // NW
    assert b_per_w % 2 == 0
    mesh = plsc.VectorSubcoreMesh(core_axis_name="c", subcore_axis_name="s")

    @functools.partial(
        pl.kernel,
        mesh=mesh,
        compiler_params=pltpu.CompilerParams(use_tc_tiling_on_sc=False),
        out_type=jax.ShapeDtypeStruct((B, DIM), jnp.float32),
        scratch_types=[
            pltpu.VMEM((b_per_w, L_PAD), jnp.int32),
            pltpu.VMEM((2, L_SEQ, DIM), jnp.float32),
            pltpu.VMEM((b_per_w, DIM), jnp.float32),
            pltpu.SemaphoreType.DMA,
            pltpu.SemaphoreType.DMA,
        ],
    )
    def pool_kernel(x_hbm, table_hbm, out_hbm, idx_v, rows_v, pooled_v,
                    sem0, sem1):
        wid = lax.axis_index("s") * NC + lax.axis_index("c")
        pltpu.sync_copy(x_hbm.at[wid], idx_v)

        sems = (sem0, sem1)

        def issue(r, b):
            for off, n in CHUNKS:
                pltpu.make_async_copy(
                    table_hbm.at[idx_v.at[r, pl.ds(off, n)]],
                    rows_v.at[b, pl.ds(off, n)],
                    sems[b],
                ).start()

        def wait(r, b):
            for off, n in CHUNKS:
                pltpu.make_async_copy(
                    table_hbm.at[idx_v.at[r, pl.ds(off, n)]],
                    rows_v.at[b, pl.ds(off, n)],
                    sems[b],
                ).wait()

        def reduce_row(r, b):
            rows = rows_v.at[b]

            def body(j, accs):
                return tuple(
                    accs[k] + rows[j, pl.ds(k * 16, 16)] for k in range(NVREG)
                )

            init = tuple(
                jnp.zeros((16,), jnp.float32) for _ in range(NVREG)
            )
            accs = lax.fori_loop(0, L_SEQ, body, init, unroll=8)
            scale = jnp.float32(1.0 / L_SEQ)
            for k in range(NVREG):
                pooled_v[r, pl.ds(k * 16, 16)] = accs[k] * scale

        issue(0, 0)

        def outer(g, carry):
            for b in range(2):
                r = g * 2 + b
                nxt = r + 1

                @pl.when(nxt < b_per_w)
                def _():
                    issue(nxt, 1 - b)

                wait(r, b)
                reduce_row(r, b)
            return carry

        lax.fori_loop(0, b_per_w // 2, outer, 0)
        pltpu.sync_copy(pooled_v, out_hbm.at[pl.ds(wid * b_per_w, b_per_w)])

    return pool_kernel


def _tail_kernel(p_ref, wt_ref, b_ref, g_ref, be_ref, o_ref):
    p = p_ref[...]
    h = jnp.dot(p, wt_ref[...], preferred_element_type=jnp.float32)
    h = h + b_ref[...]
    n = jnp.float32(1.0 / p.shape[0])
    mu = jnp.sum(h, axis=0, keepdims=True) * n
    d = h - mu
    var = jnp.sum(d * d, axis=0, keepdims=True) * n
    hn = d * lax.rsqrt(var + EPS) * g_ref[...] + be_ref[...]
    m = jnp.float32(1.0 / p.shape[1])
    mu2 = jnp.sum(hn, axis=1, keepdims=True) * m
    d2 = hn - mu2
    var2 = jnp.sum(d2 * d2, axis=1, keepdims=True) * m
    o_ref[...] = d2 * lax.rsqrt(var2 + EPS)


@jax.jit
def kernel(x, table, W, b, gamma, beta):
    B = x.shape[0]
    info = plsc.get_sparse_core_info()
    NW = info.num_cores * info.num_subcores
    x_pad = jnp.pad(x, ((0, 0), (0, L_PAD - L_SEQ)))
    x_blocks = x_pad.reshape(NW, B // NW, L_PAD)
    pooled = _make_pooling(B)(x_blocks, table)
    return pl.pallas_call(
        _tail_kernel,
        out_shape=jax.ShapeDtypeStruct((B, DIM), jnp.float32),
    )(pooled, W.T, b.reshape(1, DIM), gamma.reshape(1, DIM),
      beta.reshape(1, DIM))
